# Initial kernel scaffold; baseline (speedup 1.0000x reference)
import jax
import jax.numpy as jnp
from jax.experimental import pallas as pl


def kernel(input_ids, attention_mask, emb_table, W_cls, b_cls):
    raise NotImplementedError("write your pallas kernel here")


# speedup vs baseline: 1.1052x; 1.1052x over previous
import functools

import jax
import jax.numpy as jnp
from jax import lax
from jax.experimental import pallas as pl
from jax.experimental.pallas import tpu as pltpu
from jax.experimental.pallas import tpu_sc as plsc

B = 4096
S = 200
D = 64
C = 100
CPAD = 128

NC = 2
NS = 16
NW = NC * NS
BPW = B // NW
IDS_PER_W = BPW * S
NBUF = 4
SPLIT = 104
UNROLL = 8


def _sc_gather_sum(ids_flat, table):
    mesh = plsc.VectorSubcoreMesh(core_axis_name="c", subcore_axis_name="s")

    @functools.partial(
        pl.kernel,
        mesh=mesh,
        out_type=jax.ShapeDtypeStruct((B, D), jnp.float32),
        scratch_types=[
            pltpu.VMEM((IDS_PER_W,), jnp.int32),
            pltpu.VMEM((NBUF, S, D), jnp.float32),
            pltpu.VMEM((BPW, D), jnp.float32),
        ] + [pltpu.SemaphoreType.DMA] * NBUF,
        compiler_params=pltpu.CompilerParams(use_tc_tiling_on_sc=False),
    )
    def k(ids_hbm, tbl_hbm, out_hbm, ids_v, rows_v, acc_v, *sems):
        wid = lax.axis_index("s") * NC + lax.axis_index("c")
        base = wid * IDS_PER_W
        pltpu.sync_copy(ids_hbm.at[pl.ds(base, IDS_PER_W)], ids_v)

        def issue(r, b):
            off = pl.multiple_of(r * S, 8)
            pltpu.async_copy(
                tbl_hbm.at[ids_v.at[pl.ds(off, SPLIT)]],
                rows_v.at[b, pl.ds(0, SPLIT)], sems[b])
            off2 = pl.multiple_of(r * S + SPLIT, 8)
            pltpu.async_copy(
                tbl_hbm.at[ids_v.at[pl.ds(off2, S - SPLIT)]],
                rows_v.at[b, pl.ds(SPLIT, S - SPLIT)], sems[b])

        for b in range(NBUF):
            issue(b, b)

        def group(g, carry):
            for b in range(NBUF):
                r = g * NBUF + b
                pltpu.make_async_copy(
                    tbl_hbm.at[pl.ds(0, S)], rows_v.at[b], sems[b]).wait()

                def body(i, acc):
                    a = list(acc)
                    for u in range(UNROLL):
                        sx = i * UNROLL + u
                        for j in range(4):
                            a[j] = a[j] + rows_v[b, sx, pl.ds(j * 16, 16)]
                    return tuple(a)

                z = jnp.zeros((16,), jnp.float32)
                a0, a1, a2, a3 = lax.fori_loop(
                    0, S // UNROLL, body, (z, z, z, z))
                acc_v[r, pl.ds(0, 16)] = a0
                acc_v[r, pl.ds(16, 16)] = a1
                acc_v[r, pl.ds(32, 16)] = a2
                acc_v[r, pl.ds(48, 16)] = a3

                nxt = r + NBUF

                @pl.when(nxt < BPW)
                def _():
                    issue(nxt, b)
            return carry

        lax.fori_loop(0, BPW // NBUF, group, 0)
        pltpu.sync_copy(acc_v, out_hbm.at[pl.ds(wid * BPW, BPW)])

    return k(ids_flat, table)


def _tc_head(summed, mask, w_pad, b_pad):
    BLK = 256

    def body(s_ref, m_ref, w_ref, b_ref, o_ref):
        denom = jnp.clip(jnp.sum(m_ref[...], axis=1, keepdims=True), 1.0, None)
        pooled = s_ref[...] / denom
        o_ref[...] = lax.dot_general(
            pooled, w_ref[...], (((1,), (1,)), ((), ())),
            preferred_element_type=jnp.float32) + b_ref[...]

    return pl.pallas_call(
        body,
        grid=(B // BLK,),
        in_specs=[
            pl.BlockSpec((BLK, D), lambda i: (i, 0)),
            pl.BlockSpec((BLK, S), lambda i: (i, 0)),
            pl.BlockSpec((CPAD, D), lambda i: (0, 0)),
            pl.BlockSpec((1, CPAD), lambda i: (0, 0)),
        ],
        out_specs=pl.BlockSpec((BLK, CPAD), lambda i: (i, 0)),
        out_shape=jax.ShapeDtypeStruct((B, CPAD), jnp.float32),
    )(summed, mask, w_pad, b_pad)


def kernel(input_ids, attention_mask, emb_table, W_cls, b_cls):
    ids_flat = input_ids.reshape(-1).astype(jnp.int32)
    summed = _sc_gather_sum(ids_flat, emb_table)
    w_pad = jnp.zeros((CPAD, D), jnp.float32).at[:C].set(W_cls)
    b_pad = jnp.zeros((1, CPAD), jnp.float32).at[0, :C].set(b_cls)
    logits = _tc_head(summed, attention_mask, w_pad, b_pad)
    return logits[:, :C]
